# baseline (speedup 1.0000x reference)
import jax
import jax.numpy as jnp
from jax.experimental import pallas as pl
from jax.experimental.pallas import tpu as pltpu

N = 5000
D = 12544
H = 1024
BM = 200


def _body(x_ref, w1_ref, b1_ref, w2_ref, b2_ref, w3_ref, b3_ref,
          w4_ref, b4_ref, cls_ref, box_ref):
    h1 = jnp.maximum(
        jnp.dot(x_ref[...].astype(jnp.bfloat16), w1_ref[...],
                preferred_element_type=jnp.float32) + b1_ref[...], 0.0
    ).astype(jnp.bfloat16)
    h2 = jnp.maximum(
        jnp.dot(h1, w2_ref[...], preferred_element_type=jnp.float32)
        + b2_ref[...], 0.0).astype(jnp.bfloat16)
    logits = jnp.dot(h2, w3_ref[...],
                     preferred_element_type=jnp.float32) + b3_ref[...]
    m = jnp.max(logits, axis=-1, keepdims=True)
    e = jnp.exp(logits - m)
    cls_ref[...] = e / jnp.sum(e, axis=-1, keepdims=True)
    box_ref[...] = jnp.dot(h2, w4_ref[...],
                           preferred_element_type=jnp.float32) + b4_ref[...]


def kernel(feature_vectors, W1, b1, W2, b2, W3, b3, W4, b4):
    C1 = W3.shape[1]
    C4 = W4.shape[1]
    out = pl.pallas_call(
        _body,
        grid=(N // BM,),
        in_specs=[
            pl.BlockSpec((BM, D), lambda i: (i, 0)),
            pl.BlockSpec((D, H), lambda i: (0, 0)),
            pl.BlockSpec((1, H), lambda i: (0, 0)),
            pl.BlockSpec((H, H), lambda i: (0, 0)),
            pl.BlockSpec((1, H), lambda i: (0, 0)),
            pl.BlockSpec((H, C1), lambda i: (0, 0)),
            pl.BlockSpec((1, C1), lambda i: (0, 0)),
            pl.BlockSpec((H, C4), lambda i: (0, 0)),
            pl.BlockSpec((1, C4), lambda i: (0, 0)),
        ],
        out_specs=[
            pl.BlockSpec((BM, C1), lambda i: (i, 0)),
            pl.BlockSpec((BM, C4), lambda i: (i, 0)),
        ],
        out_shape=[
            jax.ShapeDtypeStruct((N, C1), jnp.float32),
            jax.ShapeDtypeStruct((N, C4), jnp.float32),
        ],
        compiler_params=pltpu.CompilerParams(
            dimension_semantics=("arbitrary",),
            vmem_limit_bytes=64 * 1024 * 1024,
        ),
    )(feature_vectors, W1.astype(jnp.bfloat16), b1.reshape(1, H),
      W2.astype(jnp.bfloat16), b2.reshape(1, H),
      W3.astype(jnp.bfloat16), b3.reshape(1, C1),
      W4.astype(jnp.bfloat16), b4.reshape(1, C4))
    return (out[0], out[1])
